# Initial kernel scaffold; baseline (speedup 1.0000x reference)
import jax
import jax.numpy as jnp
from jax.experimental import pallas as pl


def kernel(xyz1, xyz2):
    raise NotImplementedError("write your pallas kernel here")


# speedup vs baseline: 4.8431x; 4.8431x over previous
import jax
import jax.numpy as jnp
from jax.experimental import pallas as pl
from jax.experimental.pallas import tpu as pltpu

_K = 16
_QB = 128


def _knn_block(x1_ref, x2_ref, vals_ref, idx_ref, s_ref):
    x1 = x1_ref[0]
    x2 = x2_ref[0]
    ones_c = jnp.ones((1, x2.shape[1]), jnp.float32)
    n2 = jax.lax.dot_general(
        x2 * x2, ones_c, (((1,), (1,)), ((), ())),
        preferred_element_type=jnp.float32,
    )
    n1 = jax.lax.dot_general(
        ones_c, x1 * x1, (((1,), (1,)), ((), ())),
        preferred_element_type=jnp.float32,
    )
    d = jax.lax.dot_general(
        x2, x1, (((1,), (1,)), ((), ())),
        preferred_element_type=jnp.float32,
    )
    s_ref[...] = n2 - 2.0 * d
    krow = jax.lax.broadcasted_iota(jnp.int32, (_K, _QB), 0)

    def cond(carry):
        return carry[0] < _K

    def body(carry):
        j, vals, idxs = carry
        v = s_ref[...]
        row = jax.lax.broadcasted_iota(jnp.int32, v.shape, 0)
        m = jnp.min(v, axis=0, keepdims=True)
        ii = jnp.min(jnp.where(v == m, row, 2**30), axis=0, keepdims=True)
        s_ref[...] = jnp.where(row == ii, jnp.float32(jnp.inf), v)
        vals = jnp.where(krow == j, m, vals)
        idxs = jnp.where(krow == j, ii, idxs)
        return j + 1, vals, idxs

    _, vals, idxs = jax.lax.while_loop(
        cond, body,
        (jnp.int32(0), jnp.zeros((_K, _QB), jnp.float32),
         jnp.zeros((_K, _QB), jnp.int32)),
    )
    vals_ref[0] = jnp.sqrt(jnp.maximum(vals + n1, 0.0)).T
    idx_ref[0] = idxs.T


def kernel(xyz1, xyz2):
    B, N, C = xyz1.shape
    M = xyz2.shape[1]
    grid = (B, N // _QB)
    vals, idxs = pl.pallas_call(
        _knn_block,
        grid=grid,
        in_specs=[
            pl.BlockSpec((1, _QB, C), lambda b, i: (b, i, 0)),
            pl.BlockSpec((1, M, C), lambda b, i: (b, 0, 0)),
        ],
        out_specs=[
            pl.BlockSpec((1, _QB, _K), lambda b, i: (b, i, 0)),
            pl.BlockSpec((1, _QB, _K), lambda b, i: (b, i, 0)),
        ],
        out_shape=[
            jax.ShapeDtypeStruct((B, N, _K), jnp.float32),
            jax.ShapeDtypeStruct((B, N, _K), jnp.int32),
        ],
        scratch_shapes=[pltpu.VMEM((M, _QB), jnp.float32)],
    )(xyz1, xyz2)
    return (vals, idxs)


# speedup vs baseline: 8.7864x; 1.8142x over previous
import jax
import jax.numpy as jnp
from jax.experimental import pallas as pl
from jax.experimental.pallas import tpu as pltpu

_K = 16
_QB = 128
_NC = 256
_D = 4
_BIG = 2**30


def _extract_full(s_ref, n1, vals_ref, idx_ref):
    krow = jax.lax.broadcasted_iota(jnp.int32, (_K, _QB), 0)

    def cond(carry):
        return carry[0] < _K

    def body(carry):
        j, vals, idxs = carry
        v = s_ref[...]
        row = jax.lax.broadcasted_iota(jnp.int32, v.shape, 0)
        m = jnp.min(v, axis=0, keepdims=True)
        ii = jnp.min(jnp.where(v == m, row, _BIG), axis=0, keepdims=True)
        s_ref[...] = jnp.where(row == ii, jnp.float32(jnp.inf), v)
        vals = jnp.where(krow == j, m, vals)
        idxs = jnp.where(krow == j, ii, idxs)
        return j + 1, vals, idxs

    _, vals, idxs = jax.lax.while_loop(
        cond, body,
        (jnp.int32(0), jnp.zeros((_K, _QB), jnp.float32),
         jnp.zeros((_K, _QB), jnp.int32)),
    )
    vals_ref[0] = jnp.sqrt(jnp.maximum(vals + n1, 0.0)).T
    idx_ref[0] = idxs.T


def _knn_block(x1_ref, x2_ref, vals_ref, idx_ref, s_ref, n2_ref):
    x1 = x1_ref[0]
    x2 = x2_ref[0]
    M, C = x2.shape
    nfold = M // _NC

    @pl.when(pl.program_id(1) == 0)
    def _():
        n2_ref[...] = jax.lax.dot_general(
            x2 * x2, jnp.ones((1, C), jnp.float32), (((1,), (1,)), ((), ())),
            preferred_element_type=jnp.float32,
        )

    n1 = jax.lax.dot_general(
        jnp.ones((1, C), jnp.float32), x1 * x1, (((1,), (1,)), ((), ())),
        preferred_element_type=jnp.float32,
    )
    d = jax.lax.dot_general(
        x2, x1, (((1,), (1,)), ((), ())),
        preferred_element_type=jnp.float32,
    )
    s_ref[...] = n2_ref[...] - 2.0 * d

    inf = jnp.float32(jnp.inf)
    riota = jax.lax.broadcasted_iota(jnp.int32, (_NC, _QB), 0)
    w = [s_ref[0:_NC, :]] + [jnp.full((_NC, _QB), inf)] * (_D - 1)
    a = [riota] + [jnp.full((_NC, _QB), _BIG)] * (_D - 1)
    for f in range(1, nfold):
        x = s_ref[f * _NC:(f + 1) * _NC, :]
        xi = riota + jnp.int32(f * _NC)
        c = [x < w[t] for t in range(_D)]
        for t in range(_D - 1, 0, -1):
            w[t] = jnp.where(c[t - 1], w[t - 1], jnp.where(c[t], x, w[t]))
            a[t] = jnp.where(c[t - 1], a[t - 1], jnp.where(c[t], xi, a[t]))
        w[0] = jnp.where(c[0], x, w[0])
        a[0] = jnp.where(c[0], xi, a[0])

    krow = jax.lax.broadcasted_iota(jnp.int32, (_K, _QB), 0)

    def cond(carry):
        return carry[0] < _K

    def body(carry):
        j, w1, w2, w3, w4, a1, a2, a3, a4, vals, idxs = carry
        m = jnp.min(w1, axis=0, keepdims=True)
        io = jnp.min(jnp.where(w1 == m, a1, _BIG), axis=0, keepdims=True)
        cm = riota == (io & jnp.int32(_NC - 1))
        vals = jnp.where(krow == j, m, vals)
        idxs = jnp.where(krow == j, io, idxs)
        w1 = jnp.where(cm, w2, w1)
        w2 = jnp.where(cm, w3, w2)
        w3 = jnp.where(cm, w4, w3)
        w4 = jnp.where(cm, inf, w4)
        a1 = jnp.where(cm, a2, a1)
        a2 = jnp.where(cm, a3, a2)
        a3 = jnp.where(cm, a4, a3)
        a4 = jnp.where(cm, _BIG, a4)
        return j + 1, w1, w2, w3, w4, a1, a2, a3, a4, vals, idxs

    out = jax.lax.while_loop(
        cond, body,
        (jnp.int32(0), w[0], w[1], w[2], w[3], a[0], a[1], a[2], a[3],
         jnp.zeros((_K, _QB), jnp.float32), jnp.zeros((_K, _QB), jnp.int32)),
    )
    vals, idxs = out[9], out[10]
    vals_ref[0] = jnp.sqrt(jnp.maximum(vals + n1, 0.0)).T
    idx_ref[0] = idxs.T

    cols = idxs & jnp.int32(_NC - 1)
    cnt = jnp.zeros((1, _QB), jnp.int32)
    for i in range(_K):
        for jj in range(i + 1, _K):
            cnt = cnt + (cols[i:i + 1, :] == cols[jj:jj + 1, :]).astype(jnp.int32)

    @pl.when(jnp.any(cnt >= 6))
    def _():
        _extract_full(s_ref, n1, vals_ref, idx_ref)


def kernel(xyz1, xyz2):
    B, N, C = xyz1.shape
    M = xyz2.shape[1]
    grid = (B, N // _QB)
    vals, idxs = pl.pallas_call(
        _knn_block,
        grid=grid,
        in_specs=[
            pl.BlockSpec((1, _QB, C), lambda b, i: (b, i, 0)),
            pl.BlockSpec((1, M, C), lambda b, i: (b, 0, 0)),
        ],
        out_specs=[
            pl.BlockSpec((1, _QB, _K), lambda b, i: (b, i, 0)),
            pl.BlockSpec((1, _QB, _K), lambda b, i: (b, i, 0)),
        ],
        out_shape=[
            jax.ShapeDtypeStruct((B, N, _K), jnp.float32),
            jax.ShapeDtypeStruct((B, N, _K), jnp.int32),
        ],
        scratch_shapes=[
            pltpu.VMEM((M, _QB), jnp.float32),
            pltpu.VMEM((M, 1), jnp.float32),
        ],
    )(xyz1, xyz2)
    return (vals, idxs)


# speedup vs baseline: 15.1935x; 1.7292x over previous
import jax
import jax.numpy as jnp
from jax.experimental import pallas as pl
from jax.experimental.pallas import tpu as pltpu

_K = 16
_QB = 128
_NC = 128
_D = 4
_BIG = 2**30


def _extract_full(s_ref, n1, vals_ref, idx_ref):
    krow = jax.lax.broadcasted_iota(jnp.int32, (_K, _QB), 0)

    def cond(carry):
        return carry[0] < _K

    def body(carry):
        j, vals, idxs = carry
        v = s_ref[...]
        row = jax.lax.broadcasted_iota(jnp.int32, v.shape, 0)
        m = jnp.min(v, axis=0, keepdims=True)
        ii = jnp.min(jnp.where(v == m, row, _BIG), axis=0, keepdims=True)
        s_ref[...] = jnp.where(row == ii, jnp.float32(jnp.inf), v)
        vals = jnp.where(krow == j, m, vals)
        idxs = jnp.where(krow == j, ii, idxs)
        return j + 1, vals, idxs

    _, vals, idxs = jax.lax.while_loop(
        cond, body,
        (jnp.int32(0), jnp.zeros((_K, _QB), jnp.float32),
         jnp.zeros((_K, _QB), jnp.int32)),
    )
    vals_ref[0] = jnp.sqrt(jnp.maximum(vals + n1, 0.0)).T
    idx_ref[0] = idxs.T


def _knn_block(x1_ref, x2_ref, vals_ref, idx_ref, s_ref, n2_ref):
    x1 = x1_ref[0]
    x2 = x2_ref[0]
    M, C = x2.shape
    nfold = M // _NC

    @pl.when(pl.program_id(1) == 0)
    def _():
        n2_ref[...] = jax.lax.dot_general(
            x2 * x2, jnp.ones((1, C), jnp.float32), (((1,), (1,)), ((), ())),
            preferred_element_type=jnp.float32,
        )

    n1 = jax.lax.dot_general(
        jnp.ones((1, C), jnp.float32), x1 * x1, (((1,), (1,)), ((), ())),
        preferred_element_type=jnp.float32,
    )
    d = jax.lax.dot_general(
        x2, x1, (((1,), (1,)), ((), ())),
        preferred_element_type=jnp.float32,
    )
    s_ref[...] = n2_ref[...] - 2.0 * d

    inf = jnp.float32(jnp.inf)
    riota = jax.lax.broadcasted_iota(jnp.int32, (_NC, _QB), 0)
    w = [s_ref[0:_NC, :]] + [jnp.full((_NC, _QB), inf)] * _D
    a = [riota] + [jnp.full((_NC, _QB), _BIG)] * (_D - 1)
    for f in range(1, nfold):
        x = s_ref[f * _NC:(f + 1) * _NC, :]
        xi = riota + jnp.int32(f * _NC)
        c = [x < w[t] for t in range(_D + 1)]
        for t in range(_D, 0, -1):
            w[t] = jnp.where(c[t - 1], w[t - 1], jnp.where(c[t], x, w[t]))
            if t < _D:
                a[t] = jnp.where(c[t - 1], a[t - 1], jnp.where(c[t], xi, a[t]))
        w[0] = jnp.where(c[0], x, w[0])
        a[0] = jnp.where(c[0], xi, a[0])
    w5 = w[_D]

    krow = jax.lax.broadcasted_iota(jnp.int32, (_K, _QB), 0)

    def cond(carry):
        return carry[0] < _K

    def body(carry):
        j, w1, w2, w3, w4, a1, a2, a3, a4, vals, idxs = carry
        m = jnp.min(w1, axis=0, keepdims=True)
        io = jnp.min(jnp.where(w1 == m, a1, _BIG), axis=0, keepdims=True)
        cm = riota == (io & jnp.int32(_NC - 1))
        vals = jnp.where(krow == j, m, vals)
        idxs = jnp.where(krow == j, io, idxs)
        w1 = jnp.where(cm, w2, w1)
        w2 = jnp.where(cm, w3, w2)
        w3 = jnp.where(cm, w4, w3)
        w4 = jnp.where(cm, inf, w4)
        a1 = jnp.where(cm, a2, a1)
        a2 = jnp.where(cm, a3, a2)
        a3 = jnp.where(cm, a4, a3)
        a4 = jnp.where(cm, _BIG, a4)
        return j + 1, w1, w2, w3, w4, a1, a2, a3, a4, vals, idxs

    out = jax.lax.while_loop(
        cond, body,
        (jnp.int32(0), w[0], w[1], w[2], w[3], a[0], a[1], a[2], a[3],
         jnp.zeros((_K, _QB), jnp.float32), jnp.zeros((_K, _QB), jnp.int32)),
    )
    w1f, vals, idxs = out[1], out[9], out[10]
    vals_ref[0] = jnp.sqrt(jnp.maximum(vals + n1, 0.0)).T
    idx_ref[0] = idxs.T

    m15 = vals[_K - 1:_K, :]
    bad = (w1f == inf) & (w5 <= m15)

    @pl.when(jnp.any(bad))
    def _():
        _extract_full(s_ref, n1, vals_ref, idx_ref)


def kernel(xyz1, xyz2):
    B, N, C = xyz1.shape
    M = xyz2.shape[1]
    grid = (B, N // _QB)
    vals, idxs = pl.pallas_call(
        _knn_block,
        grid=grid,
        in_specs=[
            pl.BlockSpec((1, _QB, C), lambda b, i: (b, i, 0)),
            pl.BlockSpec((1, M, C), lambda b, i: (b, 0, 0)),
        ],
        out_specs=[
            pl.BlockSpec((1, _QB, _K), lambda b, i: (b, i, 0)),
            pl.BlockSpec((1, _QB, _K), lambda b, i: (b, i, 0)),
        ],
        out_shape=[
            jax.ShapeDtypeStruct((B, N, _K), jnp.float32),
            jax.ShapeDtypeStruct((B, N, _K), jnp.int32),
        ],
        scratch_shapes=[
            pltpu.VMEM((M, _QB), jnp.float32),
            pltpu.VMEM((M, 1), jnp.float32),
        ],
    )(xyz1, xyz2)
    return (vals, idxs)
